# baseline (device time: 21370 ns/iter reference)
import jax
import jax.numpy as jnp
from jax import lax
from jax.experimental import pallas as pl
from jax.experimental.pallas import tpu as pltpu

N_DEV = 4
B = 2
S_PER = 128
HQ = 4
DH = 64
BH = B * HQ
D_MODEL = 512
BLK = 64
SCALE = 0.125
_COMM = True


def kernel(x, Wq, K_ext, V_ext, Wo):
    def body(x_ref, wq_ref, k_ref, v_ref, wo_ref, out_ref,
             kv_all, send_sems, recv_sems):
        my = lax.axis_index("i")
        bf16 = jnp.bfloat16

        if _COMM:
            barrier_sem = pltpu.get_barrier_semaphore()
            for o in range(1, N_DEV):
                @pl.when(my - o >= 0)
                def _(o=o):
                    pl.semaphore_signal(
                        barrier_sem, inc=1,
                        device_id=(my - o,),
                        device_id_type=pl.DeviceIdType.MESH,
                    )

        for b in range(B):
            for h in range(HQ):
                kv_all[my, 0, b * HQ + h] = k_ref[b, :, h, :].astype(bf16)
                kv_all[my, 1, b * HQ + h] = v_ref[b, :, h, :].astype(bf16)

        def pair_rdma(o):
            return pltpu.make_async_remote_copy(
                src_ref=kv_all.at[my],
                dst_ref=kv_all.at[my],
                send_sem=send_sems.at[o - 1],
                recv_sem=recv_sems.at[o - 1],
                device_id=(my + o,),
                device_id_type=pl.DeviceIdType.MESH,
            )

        if _COMM:
            pl.semaphore_wait(barrier_sem, N_DEV - 1 - my)
            for o in range(1, N_DEV):
                @pl.when(my + o < N_DEV)
                def _(o=o):
                    pair_rdma(o).start()

        q16 = []
        for b in range(B):
            qf = jax.lax.dot_general(
                x_ref[b].astype(bf16), wq_ref[...].astype(bf16),
                (((1,), (0,)), ((), ())),
                preferred_element_type=jnp.float32,
            ) * SCALE
            q16.append(qf.astype(bf16))

        ib = lax.broadcasted_iota(jnp.int32, (S_PER, S_PER), 0) // BLK
        jb = lax.broadcasted_iota(jnp.int32, (S_PER, S_PER), 1) // BLK
        tri = jb <= ib

        ctx_acc = [[None] * HQ for _ in range(B)]
        den_acc = [[None] * HQ for _ in range(B)]

        for o in range(N_DEV):
            if o > 0 and _COMM:
                @pl.when(my - o >= 0)
                def _(o=o):
                    pltpu.make_async_remote_copy(
                        src_ref=kv_all.at[0],
                        dst_ref=kv_all.at[0],
                        send_sem=send_sems.at[o - 1],
                        recv_sem=recv_sems.at[o - 1],
                        device_id=(my,),
                        device_id_type=pl.DeviceIdType.MESH,
                    ).wait_recv()
            slot = jnp.maximum(my - o, 0)
            valid = (my >= o).astype(jnp.float32)
            for b in range(B):
                for h in range(HQ):
                    idx = b * HQ + h
                    q = q16[b][:, h * DH:(h + 1) * DH]
                    k_p = kv_all[slot, 0, idx]
                    v_p = kv_all[slot, 1, idx]
                    s = jax.lax.dot_general(
                        q, k_p,
                        (((1,), (1,)), ((), ())),
                        preferred_element_type=jnp.float32,
                    )
                    if o == 0:
                        s = jnp.where(tri, s, -1e9)
                    e = jnp.exp(s)
                    c = jax.lax.dot_general(
                        e.astype(bf16), v_p,
                        (((1,), (0,)), ((), ())),
                        preferred_element_type=jnp.float32,
                    )
                    d = jnp.sum(e, axis=1, keepdims=True)
                    if o == 0:
                        ctx_acc[b][h] = c
                        den_acc[b][h] = d
                    else:
                        ctx_acc[b][h] = ctx_acc[b][h] + c * valid
                        den_acc[b][h] = den_acc[b][h] + d * valid

        wo16 = wo_ref[...].astype(bf16)
        for b in range(B):
            ctx = jnp.concatenate(
                [ctx_acc[b][h] / den_acc[b][h] for h in range(HQ)], axis=1
            ).astype(bf16)
            out_ref[b] = jax.lax.dot_general(
                ctx, wo16,
                (((1,), (0,)), ((), ())),
                preferred_element_type=jnp.float32,
            )

        for o in (range(1, N_DEV) if _COMM else []):
            @pl.when(my + o < N_DEV)
            def _(o=o):
                pair_rdma(o).wait_send()

    return pl.pallas_call(
        body,
        out_shape=jax.ShapeDtypeStruct((B, S_PER, D_MODEL), jnp.float32),
        in_specs=[pl.BlockSpec(memory_space=pltpu.VMEM)] * 5,
        out_specs=pl.BlockSpec(memory_space=pltpu.VMEM),
        scratch_shapes=[
            pltpu.VMEM((N_DEV, 2, BH, S_PER, DH), jnp.bfloat16),
            pltpu.SemaphoreType.DMA((N_DEV - 1,)),
            pltpu.SemaphoreType.DMA((N_DEV - 1,)),
        ],
        compiler_params=(
            pltpu.CompilerParams(collective_id=0) if _COMM
            else pltpu.CompilerParams()
        ),
    )(x, Wq, K_ext, V_ext, Wo)
